# baseline (speedup 1.0000x reference)
import jax
import jax.numpy as jnp
from jax import lax
from jax.experimental import pallas as pl
from jax.experimental.pallas import tpu as pltpu
from jax.experimental.pallas import tpu_sc as plsc

N = 10000
E = 320000
D = 128
H = 64
G = 64
C = 2

NC = 2
NS = 16
CHUNK = 128
CHUNKS = 160
NBUF = 4
EPT = CHUNK * CHUNKS
EP = EPT * NS
ROWS_PER_TILE = 632
NPAD = ROWS_PER_TILE * NS
H2 = H // 2


def _sc_agg_body(h_hbm, src_hbm, dst_hbm, zeros_hbm, out_hbm,
                 src_idx, dst_idx, gbufs, gsems, agg_sh):
    c = lax.axis_index("c")
    s = lax.axis_index("s")
    pltpu.sync_copy(src_hbm.at[c, s], src_idx)
    pltpu.sync_copy(dst_hbm.at[s], dst_idx)
    for b in range(NBUF):
        pltpu.async_copy(h_hbm.at[src_idx.at[b]], gbufs.at[b], gsems.at[b])
    pltpu.sync_copy(zeros_hbm,
                    agg_sh.at[pl.ds(s * ROWS_PER_TILE, ROWS_PER_TILE)])
    plsc.subcore_barrier()

    def outer(jj, carry):
        for b in range(NBUF):
            j = jj * NBUF + b
            pltpu.make_async_copy(
                h_hbm.at[src_idx.at[j]], gbufs.at[b], gsems.at[b]).wait()
            pltpu.sync_copy(gbufs.at[b], agg_sh.at[dst_idx.at[j]], add=True)

            @pl.when(jj < CHUNKS // NBUF - 1)
            def _():
                pltpu.async_copy(
                    h_hbm.at[src_idx.at[j + NBUF]], gbufs.at[b], gsems.at[b])
        return carry

    lax.fori_loop(0, CHUNKS // NBUF, outer, 0)
    plsc.subcore_barrier()
    pltpu.sync_copy(agg_sh.at[pl.ds(s * ROWS_PER_TILE, ROWS_PER_TILE)],
                    out_hbm.at[c, pl.ds(s * ROWS_PER_TILE, ROWS_PER_TILE)])


def _make_sc_agg():
    mesh = plsc.VectorSubcoreMesh(core_axis_name="c", subcore_axis_name="s")
    return pl.kernel(
        _sc_agg_body,
        out_type=jax.ShapeDtypeStruct((NC, NPAD, H2), jnp.float32),
        mesh=mesh,
        compiler_params=pltpu.CompilerParams(use_tc_tiling_on_sc=False),
        scratch_types=[
            pltpu.VMEM((CHUNKS, CHUNK), jnp.int32),
            pltpu.VMEM((CHUNKS, CHUNK), jnp.int32),
            pltpu.VMEM((NBUF, CHUNK, H2), jnp.float32),
            pltpu.SemaphoreType.DMA((NBUF,)),
            pltpu.VMEM_SHARED((NPAD, H2), jnp.float32),
        ],
    )


def _tc_pre_body(x_ref, w1_ref, out_ref):
    out_ref[...] = jnp.dot(x_ref[...], w1_ref[...],
                           preferred_element_type=jnp.float32)


def _mlp_bn(g_ref, agg_ref, b1_ref, w2_ref, b2_ref, eps_ref,
            gamma_ref, beta_ref):
    g = g_ref[...]
    agg = jnp.concatenate([agg_ref[0, :N, :], agg_ref[1, :N, :]], axis=1)
    u = jnp.maximum((1.0 + eps_ref[...]) * g + agg + b1_ref[...], 0.0)
    v = jnp.dot(u, w2_ref[...], preferred_element_type=jnp.float32) + b2_ref[...]
    mean = jnp.mean(v, axis=0, keepdims=True)
    cen = v - mean
    var = jnp.mean(cen * cen, axis=0, keepdims=True)
    v = cen * lax.rsqrt(var + 1e-5) * gamma_ref[...] + beta_ref[...]
    return jnp.maximum(v, 0.0)


def _tc_layer_body(g_ref, agg_ref, b1_ref, w2_ref, b2_ref, eps_ref,
                   gamma_ref, beta_ref, w1n_ref, out_ref):
    h = _mlp_bn(g_ref, agg_ref, b1_ref, w2_ref, b2_ref, eps_ref,
                gamma_ref, beta_ref)
    out_ref[...] = jnp.dot(h, w1n_ref[...], preferred_element_type=jnp.float32)


def _tc_final_body(g_ref, agg_ref, batch_ref, b1_ref, w2_ref, b2_ref,
                   eps_ref, gamma_ref, beta_ref, wc_ref, bc_ref, out_ref):
    h = _mlp_bn(g_ref, agg_ref, b1_ref, w2_ref, b2_ref, eps_ref,
                gamma_ref, beta_ref)
    onehot = (batch_ref[...]
              == lax.broadcasted_iota(jnp.int32, (N, G), 1)).astype(jnp.float32)
    counts = jnp.sum(onehot, axis=0, keepdims=True)
    ohs = onehot * (1.0 / jnp.maximum(counts, 1.0))
    pooled = lax.dot_general(ohs, h, (((0,), (0,)), ((), ())),
                             preferred_element_type=jnp.float32)
    out_ref[...] = (jnp.dot(pooled, wc_ref[...],
                            preferred_element_type=jnp.float32) + bc_ref[...])


def kernel(x, edge_index, batch,
           W1_0, b1_0, W2_0, b2_0, eps_0, gamma_0, beta_0,
           W1_1, b1_1, W2_1, b2_1, eps_1, gamma_1, beta_1,
           W1_2, b1_2, W2_2, b2_2, eps_2, gamma_2, beta_2,
           Wc, bc):
    pad = EP - E
    order = jnp.argsort(edge_index[1])
    edge_index = edge_index[:, order]
    src2 = 2 * jnp.concatenate([edge_index[0], jnp.zeros((pad,), jnp.int32)])
    src_p = jnp.stack([src2, src2 + 1]).reshape(NC, NS, CHUNKS, CHUNK)
    dst_p = jnp.concatenate(
        [edge_index[1], jnp.full((pad,), N, jnp.int32)]).reshape(NS, CHUNKS, CHUNK)
    zeros_h2 = jnp.zeros((ROWS_PER_TILE, H2), jnp.float32)

    layers = [
        (b1_0, W2_0, b2_0, eps_0, gamma_0, beta_0),
        (b1_1, W2_1, b2_1, eps_1, gamma_1, beta_1),
        (b1_2, W2_2, b2_2, eps_2, gamma_2, beta_2),
    ]
    next_w1 = [W1_1, W1_2]

    sc_agg = _make_sc_agg()
    g = pl.pallas_call(
        _tc_pre_body,
        out_shape=jax.ShapeDtypeStruct((N, H), jnp.float32),
    )(x, W1_0)

    out = None
    for i, (b1, W2, b2, eps, gamma, beta) in enumerate(layers):
        agg = sc_agg(g.reshape(2 * N, H2), src_p, dst_p, zeros_h2)
        common = (g, agg, b1.reshape(1, H), W2, b2.reshape(1, H),
                  eps.reshape(1, 1), gamma.reshape(1, H), beta.reshape(1, H))
        if i < 2:
            g = pl.pallas_call(
                _tc_layer_body,
                out_shape=jax.ShapeDtypeStruct((N, H), jnp.float32),
            )(*common, next_w1[i])
        else:
            out = pl.pallas_call(
                _tc_final_body,
                out_shape=jax.ShapeDtypeStruct((G, C), jnp.float32),
            )(common[0], common[1], batch.reshape(N, 1), *common[2:],
              Wc, bc.reshape(1, C))
    return out


# speedup vs baseline: 1.5817x; 1.5817x over previous
import jax
import jax.numpy as jnp
from jax import lax
from jax.experimental import pallas as pl
from jax.experimental.pallas import tpu as pltpu
from jax.experimental.pallas import tpu_sc as plsc

N = 10000
E = 320000
D = 128
H = 64
G = 64
C = 2

NC = 2
NS = 16
CHUNK = 128
CHUNKS = 160
NBUF = 4
EPT = CHUNK * CHUNKS
EP = EPT * NS
ROWS_PER_TILE = 632
NPAD = ROWS_PER_TILE * NS
H2 = H // 2


def _sc_agg_body(h_hbm, src_hbm, dst_hbm, zeros_hbm, out_hbm,
                 src_idx, dst_idx, gbufs, gsems, agg_sh):
    c = lax.axis_index("c")
    s = lax.axis_index("s")
    pltpu.sync_copy(src_hbm.at[c, s], src_idx)
    pltpu.sync_copy(dst_hbm.at[s], dst_idx)
    for b in range(NBUF):
        pltpu.async_copy(h_hbm.at[src_idx.at[b]], gbufs.at[b], gsems.at[b])
    pltpu.sync_copy(zeros_hbm,
                    agg_sh.at[pl.ds(s * ROWS_PER_TILE, ROWS_PER_TILE)])
    plsc.subcore_barrier()

    def outer(jj, carry):
        for b in range(NBUF):
            j = jj * NBUF + b
            pltpu.make_async_copy(
                h_hbm.at[src_idx.at[j]], gbufs.at[b], gsems.at[b]).wait()
            pltpu.sync_copy(gbufs.at[b], agg_sh.at[dst_idx.at[j]], add=True)

            @pl.when(jj < CHUNKS // NBUF - 1)
            def _():
                pltpu.async_copy(
                    h_hbm.at[src_idx.at[j + NBUF]], gbufs.at[b], gsems.at[b])
        return carry

    lax.fori_loop(0, CHUNKS // NBUF, outer, 0)
    plsc.subcore_barrier()
    pltpu.sync_copy(agg_sh.at[pl.ds(s * ROWS_PER_TILE, ROWS_PER_TILE)],
                    out_hbm.at[c, pl.ds(s * ROWS_PER_TILE, ROWS_PER_TILE)])


def _make_sc_agg():
    mesh = plsc.VectorSubcoreMesh(core_axis_name="c", subcore_axis_name="s")
    return pl.kernel(
        _sc_agg_body,
        out_type=jax.ShapeDtypeStruct((NC, NPAD, H2), jnp.float32),
        mesh=mesh,
        compiler_params=pltpu.CompilerParams(use_tc_tiling_on_sc=False),
        scratch_types=[
            pltpu.VMEM((CHUNKS, CHUNK), jnp.int32),
            pltpu.VMEM((CHUNKS, CHUNK), jnp.int32),
            pltpu.VMEM((NBUF, CHUNK, H2), jnp.float32),
            pltpu.SemaphoreType.DMA((NBUF,)),
            pltpu.VMEM_SHARED((NPAD, H2), jnp.float32),
        ],
    )


def _tc_pre_body(x_ref, w1_ref, out_ref):
    out_ref[...] = jnp.dot(x_ref[...], w1_ref[...],
                           preferred_element_type=jnp.float32)


def _mlp_bn(g_ref, agg_ref, b1_ref, w2_ref, b2_ref, eps_ref,
            gamma_ref, beta_ref):
    g = g_ref[...]
    agg = jnp.concatenate([agg_ref[0, :N, :], agg_ref[1, :N, :]], axis=1)
    u = jnp.maximum((1.0 + eps_ref[...]) * g + agg + b1_ref[...], 0.0)
    v = jnp.dot(u, w2_ref[...], preferred_element_type=jnp.float32) + b2_ref[...]
    mean = jnp.mean(v, axis=0, keepdims=True)
    cen = v - mean
    var = jnp.mean(cen * cen, axis=0, keepdims=True)
    v = cen * lax.rsqrt(var + 1e-5) * gamma_ref[...] + beta_ref[...]
    return jnp.maximum(v, 0.0)


def _tc_layer_body(g_ref, agg_ref, b1_ref, w2_ref, b2_ref, eps_ref,
                   gamma_ref, beta_ref, w1n_ref, out_ref):
    h = _mlp_bn(g_ref, agg_ref, b1_ref, w2_ref, b2_ref, eps_ref,
                gamma_ref, beta_ref)
    out_ref[...] = jnp.dot(h, w1n_ref[...], preferred_element_type=jnp.float32)


def _tc_final_body(g_ref, agg_ref, batch_ref, b1_ref, w2_ref, b2_ref,
                   eps_ref, gamma_ref, beta_ref, wc_ref, bc_ref, out_ref):
    h = _mlp_bn(g_ref, agg_ref, b1_ref, w2_ref, b2_ref, eps_ref,
                gamma_ref, beta_ref)
    onehot = (batch_ref[...]
              == lax.broadcasted_iota(jnp.int32, (N, G), 1)).astype(jnp.float32)
    counts = jnp.sum(onehot, axis=0, keepdims=True)
    ohs = onehot * (1.0 / jnp.maximum(counts, 1.0))
    pooled = lax.dot_general(ohs, h, (((0,), (0,)), ((), ())),
                             preferred_element_type=jnp.float32)
    out_ref[...] = (jnp.dot(pooled, wc_ref[...],
                            preferred_element_type=jnp.float32) + bc_ref[...])


def kernel(x, edge_index, batch,
           W1_0, b1_0, W2_0, b2_0, eps_0, gamma_0, beta_0,
           W1_1, b1_1, W2_1, b2_1, eps_1, gamma_1, beta_1,
           W1_2, b1_2, W2_2, b2_2, eps_2, gamma_2, beta_2,
           Wc, bc):
    pad = EP - E
    src2 = 2 * jnp.concatenate([edge_index[0], jnp.zeros((pad,), jnp.int32)])
    src_p = jnp.stack([src2, src2 + 1]).reshape(NC, NS, CHUNKS, CHUNK)
    dst_p = jnp.concatenate(
        [edge_index[1], jnp.full((pad,), N, jnp.int32)]).reshape(NS, CHUNKS, CHUNK)
    zeros_h2 = jnp.zeros((ROWS_PER_TILE, H2), jnp.float32)

    layers = [
        (b1_0, W2_0, b2_0, eps_0, gamma_0, beta_0),
        (b1_1, W2_1, b2_1, eps_1, gamma_1, beta_1),
        (b1_2, W2_2, b2_2, eps_2, gamma_2, beta_2),
    ]
    next_w1 = [W1_1, W1_2]

    sc_agg = _make_sc_agg()
    g = pl.pallas_call(
        _tc_pre_body,
        out_shape=jax.ShapeDtypeStruct((N, H), jnp.float32),
    )(x, W1_0)

    out = None
    for i, (b1, W2, b2, eps, gamma, beta) in enumerate(layers):
        agg = sc_agg(g.reshape(2 * N, H2), src_p, dst_p, zeros_h2)
        common = (g, agg, b1.reshape(1, H), W2, b2.reshape(1, H),
                  eps.reshape(1, 1), gamma.reshape(1, H), beta.reshape(1, H))
        if i < 2:
            g = pl.pallas_call(
                _tc_layer_body,
                out_shape=jax.ShapeDtypeStruct((N, H), jnp.float32),
            )(*common, next_w1[i])
        else:
            out = pl.pallas_call(
                _tc_final_body,
                out_shape=jax.ShapeDtypeStruct((G, C), jnp.float32),
            )(common[0], common[1], batch.reshape(N, 1), *common[2:],
              Wc, bc.reshape(1, C))
    return out
